# baseline (speedup 1.0000x reference)
import functools

import jax
import jax.numpy as jnp
from jax import lax
from jax.experimental import pallas as pl
from jax.experimental.pallas import tpu as pltpu
from jax.experimental.pallas import tpu_sc as plsc

_L = 150
_LP = 152
_D = 768
_H = 384
_B = 16384
_RPP = 1024
_CH = 16
_NCHUNK = _RPP // _CH


def _softplus(x):
    return jnp.maximum(x, 0.0) + jnp.log1p(jnp.exp(-jnp.abs(x)))


def _make_sc_norms():
    nc = 2

    mesh = plsc.VectorSubcoreMesh(core_axis_name="c", subcore_axis_name="s")

    @functools.partial(
        pl.kernel,
        mesh=mesh,
        out_type=jax.ShapeDtypeStruct((2, _B), jnp.float32),
        scratch_types=[
            pltpu.VMEM((_L, _H), jnp.float32),
            pltpu.VMEM((_RPP,), jnp.int32),
            pltpu.VMEM((_CH, _H), jnp.float32),
            pltpu.VMEM((1, _RPP), jnp.float32),
        ],
        compiler_params=pltpu.CompilerParams(
            use_tc_tiling_on_sc=False,
            needs_layout_passes=False,
        ),
    )
    def sc_norms(x_hbm, cent_hbm, lab_hbm, out_hbm, cent_v, lab_v, buf, out_v):
        wid = lax.axis_index("s") * nc + lax.axis_index("c")
        pair = wid // 2
        half = wid % 2
        rb = pair * _RPP
        fb = half * _H

        pltpu.sync_copy(lab_hbm.at[pl.ds(rb, _RPP)], lab_v)
        pltpu.sync_copy(cent_hbm.at[:, pl.ds(fb, _H)], cent_v)

        iota16 = lax.iota(jnp.int32, 16)

        def chunk_body(g, _):
            pltpu.sync_copy(
                x_hbm.at[pl.ds(rb + g * _CH, _CH), pl.ds(fb, _H)], buf)
            lv = lab_v[pl.ds(g * _CH, 16)]

            def fbody(j, acc):
                for t in range(8):
                    f = j * 8 + t
                    fcol = jnp.full((16,), f, jnp.int32)
                    xv = plsc.load_gather(buf, [iota16, fcol])
                    cv = plsc.load_gather(cent_v, [lv, fcol])
                    dd = xv - cv
                    acc = acc + dd * dd
                return acc

            acc = lax.fori_loop(0, _H // 8, fbody,
                                jnp.zeros((16,), jnp.float32))
            out_v[0, pl.ds(g * _CH, 16)] = acc
            return 0

        lax.fori_loop(0, _NCHUNK, chunk_body, 0)
        pltpu.sync_copy(out_v, out_hbm.at[pl.ds(half, 1), pl.ds(rb, _RPP)])

    return sc_norms


_EBS = 2048


def _epi_body(w_ref, lab_ref, tab_ref, drow_ref, s_ref,
              loss_ref, dsp_ref, acc_ref):
    i = pl.program_id(0)
    nb = pl.num_programs(0)

    @pl.when(i == 0)
    def _init():
        acc_ref[0] = 0.0
        acc_ref[1] = 0.0
        dsp_ref[...] = _softplus(drow_ref[...])

    lab = lab_ref[pl.ds(i, 1), :]
    iota = lax.broadcasted_iota(jnp.int32, (_LP, _EBS), 0)
    oh_t = (iota == lab).astype(jnp.float32)
    gt = lax.dot_general(tab_ref[...], oh_t,
                         (((0,), (0,)), ((), ())),
                         preferred_element_type=jnp.float32)
    k1 = _softplus(gt[0:1, :])
    k2 = _softplus(gt[1:2, :])
    d = _softplus(gt[2:3, :])

    s = s_ref[...]
    z1 = jnp.sqrt(s[0:1, :])
    z2 = jnp.sqrt(s[1:2, :])
    euc = z1 * k1 + z2 * k2
    acc_ref[0] += jnp.sum(jnp.maximum(euc - d, 0.0))
    acc_ref[1] += jnp.sum(jnp.maximum(d - euc, 0.0))

    @pl.when(i == nb - 1)
    def _fin():
        loss_ref[0, 0] = (w_ref[0, 0] * acc_ref[0] + acc_ref[1]) / (nb * _EBS)


def _epilogue(s_arr, labels2d, tab, drow, w_arr):
    nb = _B // _EBS
    return pl.pallas_call(
        _epi_body,
        grid=(nb,),
        in_specs=[
            pl.BlockSpec(memory_space=pltpu.SMEM),
            pl.BlockSpec((nb, _EBS), lambda i: (0, 0)),
            pl.BlockSpec((_LP, 8), lambda i: (0, 0)),
            pl.BlockSpec((1, _LP), lambda i: (0, 0)),
            pl.BlockSpec((2, _EBS), lambda i: (0, i)),
        ],
        out_specs=[
            pl.BlockSpec(memory_space=pltpu.SMEM),
            pl.BlockSpec((1, _LP), lambda i: (0, 0)),
        ],
        out_shape=[
            jax.ShapeDtypeStruct((1, 1), jnp.float32),
            jax.ShapeDtypeStruct((1, _LP), jnp.float32),
        ],
        scratch_shapes=[pltpu.SMEM((2,), jnp.float32)],
        compiler_params=pltpu.CompilerParams(
            dimension_semantics=("arbitrary",),
        ),
    )(w_arr, labels2d, tab, drow, s_arr)


def kernel(pooled_output, centroids, labels, delta, param_ab, w=1.0):
    labels = labels.astype(jnp.int32)
    sc_norms = _make_sc_norms()
    s_arr = sc_norms(pooled_output, centroids, labels)

    tab = jnp.zeros((_LP, 8), jnp.float32)
    tab = tab.at[:_L, 0].set(param_ab[:, 0])
    tab = tab.at[:_L, 1].set(param_ab[:, 1])
    tab = tab.at[:_L, 2].set(delta)
    drow = jnp.zeros((1, _LP), jnp.float32).at[0, :_L].set(delta)
    lab2d = labels.reshape(_B // _EBS, _EBS)
    w_arr = jnp.asarray(w, jnp.float32).reshape(1, 1)

    loss, dsp_row = _epilogue(s_arr, lab2d, tab, drow, w_arr)
    return loss[0, 0], dsp_row[0, :_L]


# speedup vs baseline: 1.9858x; 1.9858x over previous
import functools

import jax
import jax.numpy as jnp
from jax import lax
from jax.experimental import pallas as pl
from jax.experimental.pallas import tpu as pltpu
from jax.experimental.pallas import tpu_sc as plsc

_L = 150
_LP = 152
_D = 768
_H = 384
_B = 16384
_RPP = 1024
_CH = 16
_NCHUNK = _RPP // _CH


def _softplus(x):
    return jnp.maximum(x, 0.0) + jnp.log1p(jnp.exp(-jnp.abs(x)))


def _make_sc_norms():
    import numpy as np
    nc = 2

    mesh = plsc.VectorSubcoreMesh(core_axis_name="c", subcore_axis_name="s")

    @functools.partial(
        pl.kernel,
        mesh=mesh,
        out_type=jax.ShapeDtypeStruct((2, _B), jnp.float32),
        scratch_types=[
            pltpu.VMEM((_L, _H), jnp.float32),
            pltpu.VMEM((_RPP,), jnp.int32),
            pltpu.VMEM((_CH, _H), jnp.float32),
            pltpu.VMEM((_CH, _H), jnp.float32),
            pltpu.VMEM((1, _RPP), jnp.float32),
            pltpu.SemaphoreType.DMA,
            pltpu.SemaphoreType.DMA,
        ],
        compiler_params=pltpu.CompilerParams(
            use_tc_tiling_on_sc=False,
            needs_layout_passes=False,
        ),
    )
    def sc_norms(x_hbm, cent_hbm, lab_hbm, out_hbm,
                 cent_v, lab_v, buf0, buf1, out_v, sem0, sem1):
        wid = lax.axis_index("s") * nc + lax.axis_index("c")
        pair = wid // 2
        half = wid % 2
        rb = pair * _RPP
        fb = half * _H

        pltpu.sync_copy(lab_hbm.at[pl.ds(rb, _RPP)], lab_v)
        pltpu.sync_copy(cent_hbm.at[:, pl.ds(fb, _H)], cent_v)

        def x_slice(g):
            return x_hbm.at[pl.ds(rb + g * _CH, _CH), pl.ds(fb, _H)]

        iota16 = lax.iota(jnp.int32, 16)
        col_idx = [iota16 + 16 * j for j in range(_H // 16)]
        row_sel = [iota16 == r for r in range(16)]
        bcast = [(iota16 & 0) + r for r in range(16)]
        swaps = [iota16 ^ sh for sh in (8, 4, 2, 1)]

        def compute_chunk(buf, g):
            lv = lab_v[pl.ds(g * _CH, 16)]
            srow = jnp.zeros((16,), jnp.float32)
            for r in range(_CH):
                lab_r = lv.at[bcast[r]].get(mode="promise_in_bounds")
                acc = jnp.zeros((16,), jnp.float32)
                for j in range(_H // 16):
                    xv = buf[r, pl.ds(j * 16, 16)]
                    cv = plsc.load_gather(cent_v, [lab_r, col_idx[j]])
                    dd = xv - cv
                    acc = acc + dd * dd
                for sw in swaps:
                    acc = acc + acc.at[sw].get(mode="promise_in_bounds")
                srow = jnp.where(row_sel[r], acc, srow)
            out_v[0, pl.ds(g * _CH, 16)] = srow

        pltpu.async_copy(x_slice(0), buf0, sem0)
        pltpu.async_copy(x_slice(1), buf1, sem1)

        def body(i, _):
            c0 = 2 * i
            pltpu.make_async_copy(x_slice(0), buf0, sem0).wait()
            compute_chunk(buf0, c0)
            pltpu.async_copy(
                x_slice(jnp.minimum(c0 + 2, _NCHUNK - 1)), buf0, sem0)
            c1 = 2 * i + 1
            pltpu.make_async_copy(x_slice(0), buf1, sem1).wait()
            compute_chunk(buf1, c1)
            pltpu.async_copy(
                x_slice(jnp.minimum(c1 + 2, _NCHUNK - 1)), buf1, sem1)
            return 0

        lax.fori_loop(0, _NCHUNK // 2, body, 0)
        pltpu.make_async_copy(x_slice(0), buf0, sem0).wait()
        pltpu.make_async_copy(x_slice(0), buf1, sem1).wait()

        pltpu.sync_copy(out_v, out_hbm.at[pl.ds(half, 1), pl.ds(rb, _RPP)])

    return sc_norms


_EBS = 2048


def _epi_body(w_ref, lab_ref, tab_ref, drow_ref, s_ref,
              loss_ref, dsp_ref, acc_ref):
    i = pl.program_id(0)
    nb = pl.num_programs(0)

    @pl.when(i == 0)
    def _init():
        acc_ref[0] = 0.0
        acc_ref[1] = 0.0
        dsp_ref[...] = _softplus(drow_ref[...])

    lab = lab_ref[pl.ds(i, 1), :]
    iota = lax.broadcasted_iota(jnp.int32, (_LP, _EBS), 0)
    oh_t = (iota == lab).astype(jnp.float32)
    gt = lax.dot_general(tab_ref[...], oh_t,
                         (((0,), (0,)), ((), ())),
                         preferred_element_type=jnp.float32)
    k1 = _softplus(gt[0:1, :])
    k2 = _softplus(gt[1:2, :])
    d = _softplus(gt[2:3, :])

    s = s_ref[...]
    z1 = jnp.sqrt(s[0:1, :])
    z2 = jnp.sqrt(s[1:2, :])
    euc = z1 * k1 + z2 * k2
    acc_ref[0] += jnp.sum(jnp.maximum(euc - d, 0.0))
    acc_ref[1] += jnp.sum(jnp.maximum(d - euc, 0.0))

    @pl.when(i == nb - 1)
    def _fin():
        loss_ref[0, 0] = (w_ref[0, 0] * acc_ref[0] + acc_ref[1]) / (nb * _EBS)


def _epilogue(s_arr, labels2d, tab, drow, w_arr):
    nb = _B // _EBS
    return pl.pallas_call(
        _epi_body,
        grid=(nb,),
        in_specs=[
            pl.BlockSpec(memory_space=pltpu.SMEM),
            pl.BlockSpec((nb, _EBS), lambda i: (0, 0)),
            pl.BlockSpec((_LP, 8), lambda i: (0, 0)),
            pl.BlockSpec((1, _LP), lambda i: (0, 0)),
            pl.BlockSpec((2, _EBS), lambda i: (0, i)),
        ],
        out_specs=[
            pl.BlockSpec(memory_space=pltpu.SMEM),
            pl.BlockSpec((1, _LP), lambda i: (0, 0)),
        ],
        out_shape=[
            jax.ShapeDtypeStruct((1, 1), jnp.float32),
            jax.ShapeDtypeStruct((1, _LP), jnp.float32),
        ],
        scratch_shapes=[pltpu.SMEM((2,), jnp.float32)],
        compiler_params=pltpu.CompilerParams(
            dimension_semantics=("arbitrary",),
        ),
    )(w_arr, labels2d, tab, drow, s_arr)


def kernel(pooled_output, centroids, labels, delta, param_ab, w=1.0):
    labels = labels.astype(jnp.int32)
    sc_norms = _make_sc_norms()
    s_arr = sc_norms(pooled_output, centroids, labels)

    tab = jnp.zeros((_LP, 8), jnp.float32)
    tab = tab.at[:_L, 0].set(param_ab[:, 0])
    tab = tab.at[:_L, 1].set(param_ab[:, 1])
    tab = tab.at[:_L, 2].set(delta)
    drow = jnp.zeros((1, _LP), jnp.float32).at[0, :_L].set(delta)
    lab2d = labels.reshape(_B // _EBS, _EBS)
    w_arr = jnp.asarray(w, jnp.float32).reshape(1, 1)

    loss, dsp_row = _epilogue(s_arr, lab2d, tab, drow, w_arr)
    return loss[0, 0], dsp_row[0, :_L]


# speedup vs baseline: 3.7943x; 1.9107x over previous
import functools

import jax
import jax.numpy as jnp
from jax import lax
from jax.experimental import pallas as pl
from jax.experimental.pallas import tpu as pltpu
from jax.experimental.pallas import tpu_sc as plsc

_L = 150
_LP = 152
_D = 768
_H = 384
_B = 16384
_RPP = 1024
_CH = 16
_NCHUNK = _RPP // _CH


def _softplus(x):
    return jnp.maximum(x, 0.0) + jnp.log1p(jnp.exp(-jnp.abs(x)))


def _make_sc_norms():
    import numpy as np
    nc = 2

    mesh = plsc.VectorSubcoreMesh(core_axis_name="c", subcore_axis_name="s")

    @functools.partial(
        pl.kernel,
        mesh=mesh,
        out_type=jax.ShapeDtypeStruct((2, _B), jnp.float32),
        scratch_types=[
            pltpu.VMEM((_L, _H), jnp.float32),
            pltpu.VMEM((_RPP,), jnp.int32),
            pltpu.VMEM((_CH, _H), jnp.float32),
            pltpu.VMEM((_CH, _H), jnp.float32),
            pltpu.VMEM((1, _RPP), jnp.float32),
            pltpu.SemaphoreType.DMA,
            pltpu.SemaphoreType.DMA,
        ],
        compiler_params=pltpu.CompilerParams(
            use_tc_tiling_on_sc=False,
            needs_layout_passes=False,
        ),
    )
    def sc_norms(x_hbm, cent_hbm, lab_hbm, out_hbm,
                 cent_v, lab_v, buf0, buf1, out_v, sem0, sem1):
        wid = lax.axis_index("s") * nc + lax.axis_index("c")
        pair = wid // 2
        half = wid % 2
        rb = pair * _RPP
        fb = half * _H

        pltpu.sync_copy(lab_hbm.at[pl.ds(rb, _RPP)], lab_v)
        pltpu.sync_copy(cent_hbm.at[:, pl.ds(fb, _H)], cent_v)

        def x_slice(g):
            return x_hbm.at[pl.ds(rb + g * _CH, _CH), pl.ds(fb, _H)]

        iota16 = lax.iota(jnp.int32, 16)

        def compute_chunk(buf, g):
            lv = lab_v[pl.ds(g * _CH, 16)]
            phi = iota16
            zero = jnp.zeros((16,), jnp.float32)
            accs = [zero, zero, zero, zero]

            def blk(t, carry):
                phi, a0, a1, a2, a3 = carry
                accs = [a0, a1, a2, a3]
                for u in range(16):
                    xv = plsc.load_gather(buf, [iota16, phi])
                    cv = plsc.load_gather(cent_v, [lv, phi])
                    dd = xv - cv
                    accs[u % 4] = accs[u % 4] + dd * dd
                    phi = phi + 1
                return (phi, *accs)

            nblk = (_H - 16) // 16
            phi, *accs = lax.fori_loop(0, nblk, blk, (phi, *accs))
            for u in range(16):
                xv = plsc.load_gather(buf, [iota16, phi])
                cv = plsc.load_gather(cent_v, [lv, phi])
                dd = xv - cv
                accs[u % 4] = accs[u % 4] + dd * dd
                phi1 = phi + 1
                phi = jnp.where(phi1 == _H, 0, phi1)
            s = (accs[0] + accs[1]) + (accs[2] + accs[3])
            out_v[0, pl.ds(g * _CH, 16)] = s

        pltpu.async_copy(x_slice(0), buf0, sem0)
        pltpu.async_copy(x_slice(1), buf1, sem1)

        def body(i, _):
            c0 = 2 * i
            pltpu.make_async_copy(x_slice(0), buf0, sem0).wait()
            compute_chunk(buf0, c0)
            pltpu.async_copy(
                x_slice(jnp.minimum(c0 + 2, _NCHUNK - 1)), buf0, sem0)
            c1 = 2 * i + 1
            pltpu.make_async_copy(x_slice(0), buf1, sem1).wait()
            compute_chunk(buf1, c1)
            pltpu.async_copy(
                x_slice(jnp.minimum(c1 + 2, _NCHUNK - 1)), buf1, sem1)
            return 0

        lax.fori_loop(0, _NCHUNK // 2, body, 0)
        pltpu.make_async_copy(x_slice(0), buf0, sem0).wait()
        pltpu.make_async_copy(x_slice(0), buf1, sem1).wait()

        pltpu.sync_copy(out_v, out_hbm.at[pl.ds(half, 1), pl.ds(rb, _RPP)])

    return sc_norms


_EBS = 2048


def _epi_body(w_ref, lab_ref, tab_ref, drow_ref, s_ref,
              loss_ref, dsp_ref, acc_ref):
    i = pl.program_id(0)
    nb = pl.num_programs(0)

    @pl.when(i == 0)
    def _init():
        acc_ref[0] = 0.0
        acc_ref[1] = 0.0
        dsp_ref[...] = _softplus(drow_ref[...])

    lab = lab_ref[pl.ds(i, 1), :]
    iota = lax.broadcasted_iota(jnp.int32, (_LP, _EBS), 0)
    oh_t = (iota == lab).astype(jnp.float32)
    gt = lax.dot_general(tab_ref[...], oh_t,
                         (((0,), (0,)), ((), ())),
                         preferred_element_type=jnp.float32)
    k1 = _softplus(gt[0:1, :])
    k2 = _softplus(gt[1:2, :])
    d = _softplus(gt[2:3, :])

    s = s_ref[...]
    z1 = jnp.sqrt(s[0:1, :])
    z2 = jnp.sqrt(s[1:2, :])
    euc = z1 * k1 + z2 * k2
    acc_ref[0] += jnp.sum(jnp.maximum(euc - d, 0.0))
    acc_ref[1] += jnp.sum(jnp.maximum(d - euc, 0.0))

    @pl.when(i == nb - 1)
    def _fin():
        loss_ref[0, 0] = (w_ref[0, 0] * acc_ref[0] + acc_ref[1]) / (nb * _EBS)


def _epilogue(s_arr, labels2d, tab, drow, w_arr):
    nb = _B // _EBS
    return pl.pallas_call(
        _epi_body,
        grid=(nb,),
        in_specs=[
            pl.BlockSpec(memory_space=pltpu.SMEM),
            pl.BlockSpec((nb, _EBS), lambda i: (0, 0)),
            pl.BlockSpec((_LP, 8), lambda i: (0, 0)),
            pl.BlockSpec((1, _LP), lambda i: (0, 0)),
            pl.BlockSpec((2, _EBS), lambda i: (0, i)),
        ],
        out_specs=[
            pl.BlockSpec(memory_space=pltpu.SMEM),
            pl.BlockSpec((1, _LP), lambda i: (0, 0)),
        ],
        out_shape=[
            jax.ShapeDtypeStruct((1, 1), jnp.float32),
            jax.ShapeDtypeStruct((1, _LP), jnp.float32),
        ],
        scratch_shapes=[pltpu.SMEM((2,), jnp.float32)],
        compiler_params=pltpu.CompilerParams(
            dimension_semantics=("arbitrary",),
        ),
    )(w_arr, labels2d, tab, drow, s_arr)


def kernel(pooled_output, centroids, labels, delta, param_ab, w=1.0):
    labels = labels.astype(jnp.int32)
    sc_norms = _make_sc_norms()
    s_arr = sc_norms(pooled_output, centroids, labels)

    tab = jnp.zeros((_LP, 8), jnp.float32)
    tab = tab.at[:_L, 0].set(param_ab[:, 0])
    tab = tab.at[:_L, 1].set(param_ab[:, 1])
    tab = tab.at[:_L, 2].set(delta)
    drow = jnp.zeros((1, _LP), jnp.float32).at[0, :_L].set(delta)
    lab2d = labels.reshape(_B // _EBS, _EBS)
    w_arr = jnp.asarray(w, jnp.float32).reshape(1, 1)

    loss, dsp_row = _epilogue(s_arr, lab2d, tab, drow, w_arr)
    return loss[0, 0], dsp_row[0, :_L]


# speedup vs baseline: 5.6472x; 1.4883x over previous
import functools

import jax
import jax.numpy as jnp
from jax import lax
from jax.experimental import pallas as pl
from jax.experimental.pallas import tpu as pltpu
from jax.experimental.pallas import tpu_sc as plsc

_L = 150
_LP = 152
_D = 768
_H = 384
_B = 16384
_RPP = 1024
_CH = 16
_NCHUNK = _RPP // _CH


def _softplus(x):
    return jnp.maximum(x, 0.0) + jnp.log1p(jnp.exp(-jnp.abs(x)))


def _make_sc_norms():
    import numpy as np
    nc = 2

    mesh = plsc.VectorSubcoreMesh(core_axis_name="c", subcore_axis_name="s")

    @functools.partial(
        pl.kernel,
        mesh=mesh,
        out_type=jax.ShapeDtypeStruct((2, _B), jnp.float32),
        scratch_types=[
            pltpu.VMEM((_L, _H), jnp.float32),
            pltpu.VMEM((_RPP,), jnp.int32),
            pltpu.VMEM((_CH, _H), jnp.float32),
            pltpu.VMEM((_CH, _H), jnp.float32),
            pltpu.VMEM((1, _RPP), jnp.float32),
            pltpu.SemaphoreType.DMA,
            pltpu.SemaphoreType.DMA,
        ],
        compiler_params=pltpu.CompilerParams(
            use_tc_tiling_on_sc=True,
            needs_layout_passes=False,
        ),
    )
    def sc_norms(x_hbm, cent_hbm, lab_hbm, out_hbm,
                 cent_v, lab_v, buf0, buf1, out_v, sem0, sem1):
        wid = lax.axis_index("s") * nc + lax.axis_index("c")
        pair = wid // 2
        half = wid % 2
        rb = pair * _RPP
        fb = half * _H

        pltpu.sync_copy(lab_hbm.at[pl.ds(rb, _RPP)], lab_v)
        pltpu.sync_copy(cent_hbm.at[:, pl.ds(fb, _H)], cent_v)

        def x_slice(g):
            return x_hbm.at[pl.ds(rb + g * _CH, _CH), pl.ds(fb, _H)]

        iota16 = lax.iota(jnp.int32, 16)

        def compute_chunk(buf, g):
            lv = lab_v[pl.ds(g * _CH, 16)]
            phi = iota16
            zero = jnp.zeros((16,), jnp.float32)
            accs = [zero, zero, zero, zero]

            def blk(t, carry):
                phi, a0, a1, a2, a3 = carry
                accs = [a0, a1, a2, a3]
                for u in range(16):
                    xv = plsc.load_gather(buf, [iota16, phi])
                    cv = plsc.load_gather(cent_v, [lv, phi])
                    dd = xv - cv
                    accs[u % 4] = accs[u % 4] + dd * dd
                    phi = phi + 1
                return (phi, *accs)

            nblk = (_H - 16) // 16
            phi, *accs = lax.fori_loop(0, nblk, blk, (phi, *accs))
            for u in range(16):
                xv = plsc.load_gather(buf, [iota16, phi])
                cv = plsc.load_gather(cent_v, [lv, phi])
                dd = xv - cv
                accs[u % 4] = accs[u % 4] + dd * dd
                phi1 = phi + 1
                phi = jnp.where(phi1 == _H, 0, phi1)
            s = (accs[0] + accs[1]) + (accs[2] + accs[3])
            out_v[0, pl.ds(g * _CH, 16)] = s

        pltpu.async_copy(x_slice(0), buf0, sem0)
        pltpu.async_copy(x_slice(1), buf1, sem1)

        def body(i, _):
            c0 = 2 * i
            pltpu.make_async_copy(x_slice(0), buf0, sem0).wait()
            compute_chunk(buf0, c0)
            pltpu.async_copy(
                x_slice(jnp.minimum(c0 + 2, _NCHUNK - 1)), buf0, sem0)
            c1 = 2 * i + 1
            pltpu.make_async_copy(x_slice(0), buf1, sem1).wait()
            compute_chunk(buf1, c1)
            pltpu.async_copy(
                x_slice(jnp.minimum(c1 + 2, _NCHUNK - 1)), buf1, sem1)
            return 0

        lax.fori_loop(0, _NCHUNK // 2, body, 0)
        pltpu.make_async_copy(x_slice(0), buf0, sem0).wait()
        pltpu.make_async_copy(x_slice(0), buf1, sem1).wait()

        pltpu.sync_copy(out_v, out_hbm.at[pl.ds(half, 1), pl.ds(rb, _RPP)])

    return sc_norms


_EBS = 2048


def _epi_body(w_ref, lab_ref, tab_ref, drow_ref, s_ref,
              loss_ref, dsp_ref, acc_ref):
    i = pl.program_id(0)
    nb = pl.num_programs(0)

    @pl.when(i == 0)
    def _init():
        acc_ref[0] = 0.0
        acc_ref[1] = 0.0
        dsp_ref[...] = _softplus(drow_ref[...])

    lab = lab_ref[pl.ds(i, 1), :]
    iota = lax.broadcasted_iota(jnp.int32, (_LP, _EBS), 0)
    oh_t = (iota == lab).astype(jnp.float32)
    gt = lax.dot_general(tab_ref[...], oh_t,
                         (((0,), (0,)), ((), ())),
                         preferred_element_type=jnp.float32)
    k1 = _softplus(gt[0:1, :])
    k2 = _softplus(gt[1:2, :])
    d = _softplus(gt[2:3, :])

    s = s_ref[...]
    z1 = jnp.sqrt(s[0:1, :])
    z2 = jnp.sqrt(s[1:2, :])
    euc = z1 * k1 + z2 * k2
    acc_ref[0] += jnp.sum(jnp.maximum(euc - d, 0.0))
    acc_ref[1] += jnp.sum(jnp.maximum(d - euc, 0.0))

    @pl.when(i == nb - 1)
    def _fin():
        loss_ref[0, 0] = (w_ref[0, 0] * acc_ref[0] + acc_ref[1]) / (nb * _EBS)


def _epilogue(s_arr, labels2d, tab, drow, w_arr):
    nb = _B // _EBS
    return pl.pallas_call(
        _epi_body,
        grid=(nb,),
        in_specs=[
            pl.BlockSpec(memory_space=pltpu.SMEM),
            pl.BlockSpec((nb, _EBS), lambda i: (0, 0)),
            pl.BlockSpec((_LP, 8), lambda i: (0, 0)),
            pl.BlockSpec((1, _LP), lambda i: (0, 0)),
            pl.BlockSpec((2, _EBS), lambda i: (0, i)),
        ],
        out_specs=[
            pl.BlockSpec(memory_space=pltpu.SMEM),
            pl.BlockSpec((1, _LP), lambda i: (0, 0)),
        ],
        out_shape=[
            jax.ShapeDtypeStruct((1, 1), jnp.float32),
            jax.ShapeDtypeStruct((1, _LP), jnp.float32),
        ],
        scratch_shapes=[pltpu.SMEM((2,), jnp.float32)],
        compiler_params=pltpu.CompilerParams(
            dimension_semantics=("arbitrary",),
        ),
    )(w_arr, labels2d, tab, drow, s_arr)


def kernel(pooled_output, centroids, labels, delta, param_ab, w=1.0):
    labels = labels.astype(jnp.int32)
    sc_norms = _make_sc_norms()
    s_arr = sc_norms(pooled_output, centroids, labels)

    tab = jnp.zeros((_LP, 8), jnp.float32)
    tab = tab.at[:_L, 0].set(param_ab[:, 0])
    tab = tab.at[:_L, 1].set(param_ab[:, 1])
    tab = tab.at[:_L, 2].set(delta)
    drow = jnp.zeros((1, _LP), jnp.float32).at[0, :_L].set(delta)
    lab2d = labels.reshape(_B // _EBS, _EBS)
    w_arr = jnp.asarray(w, jnp.float32).reshape(1, 1)

    loss, dsp_row = _epilogue(s_arr, lab2d, tab, drow, w_arr)
    return loss[0, 0], dsp_row[0, :_L]


# speedup vs baseline: 5.7895x; 1.0252x over previous
import functools

import jax
import jax.numpy as jnp
from jax import lax
from jax.experimental import pallas as pl
from jax.experimental.pallas import tpu as pltpu
from jax.experimental.pallas import tpu_sc as plsc

_L = 150
_LP = 152
_D = 768
_H = 384
_B = 16384
_RPP = 1024
_CH = 16
_NCHUNK = _RPP // _CH


def _softplus(x):
    return jnp.maximum(x, 0.0) + jnp.log1p(jnp.exp(-jnp.abs(x)))


def _make_sc_norms():
    import numpy as np
    nc = 2

    mesh = plsc.VectorSubcoreMesh(core_axis_name="c", subcore_axis_name="s")

    @functools.partial(
        pl.kernel,
        mesh=mesh,
        out_type=jax.ShapeDtypeStruct((2, _B), jnp.float32),
        scratch_types=[
            pltpu.VMEM((_L * _H,), jnp.float32),
            pltpu.VMEM((_RPP,), jnp.int32),
            pltpu.VMEM((_CH, _H), jnp.float32),
            pltpu.VMEM((_CH, _H), jnp.float32),
            pltpu.VMEM((1, _RPP), jnp.float32),
            pltpu.SemaphoreType.DMA,
            pltpu.SemaphoreType.DMA,
        ],
        compiler_params=pltpu.CompilerParams(
            use_tc_tiling_on_sc=True,
            needs_layout_passes=False,
        ),
    )
    def sc_norms(x_hbm, cent_hbm, lab_hbm, out_hbm,
                 cent_v, lab_v, buf0, buf1, out_v, sem0, sem1):
        wid = lax.axis_index("s") * nc + lax.axis_index("c")
        pair = wid // 2
        half = wid % 2
        rb = pair * _RPP
        fb = half * _H

        pltpu.sync_copy(lab_hbm.at[pl.ds(rb, _RPP)], lab_v)
        pltpu.sync_copy(cent_hbm.at[pl.ds(half * _L * _H, _L * _H)], cent_v)

        def x_slice(g):
            return x_hbm.at[pl.ds(rb + g * _CH, _CH), pl.ds(fb, _H)]

        iota16 = lax.iota(jnp.int32, 16)

        def compute_chunk(buf, g):
            lv = lab_v[pl.ds(g * _CH, 16)]
            phi = iota16
            cptr = lv * _H + iota16
            zero = jnp.zeros((16,), jnp.float32)
            accs = [zero, zero, zero, zero]

            def blk(t, carry):
                phi, cptr, a0, a1, a2, a3 = carry
                accs = [a0, a1, a2, a3]
                for u in range(16):
                    xv = plsc.load_gather(buf, [iota16, phi])
                    cv = plsc.load_gather(cent_v, [cptr])
                    dd = xv - cv
                    accs[u % 4] = accs[u % 4] + dd * dd
                    phi = phi + 1
                    cptr = cptr + 1
                return (phi, cptr, *accs)

            nblk = (_H - 16) // 16
            phi, cptr, *accs = lax.fori_loop(0, nblk, blk, (phi, cptr, *accs))
            for u in range(16):
                xv = plsc.load_gather(buf, [iota16, phi])
                cv = plsc.load_gather(cent_v, [cptr])
                dd = xv - cv
                accs[u % 4] = accs[u % 4] + dd * dd
                phi1 = phi + 1
                wrap = phi1 == _H
                phi = jnp.where(wrap, 0, phi1)
                cptr = jnp.where(wrap, cptr + 1 - _H, cptr + 1)
            s = (accs[0] + accs[1]) + (accs[2] + accs[3])
            out_v[0, pl.ds(g * _CH, 16)] = s

        pltpu.async_copy(x_slice(0), buf0, sem0)
        pltpu.async_copy(x_slice(1), buf1, sem1)

        def body(i, _):
            c0 = 2 * i
            pltpu.make_async_copy(x_slice(0), buf0, sem0).wait()
            compute_chunk(buf0, c0)
            pltpu.async_copy(
                x_slice(jnp.minimum(c0 + 2, _NCHUNK - 1)), buf0, sem0)
            c1 = 2 * i + 1
            pltpu.make_async_copy(x_slice(0), buf1, sem1).wait()
            compute_chunk(buf1, c1)
            pltpu.async_copy(
                x_slice(jnp.minimum(c1 + 2, _NCHUNK - 1)), buf1, sem1)
            return 0

        lax.fori_loop(0, _NCHUNK // 2, body, 0)
        pltpu.make_async_copy(x_slice(0), buf0, sem0).wait()
        pltpu.make_async_copy(x_slice(0), buf1, sem1).wait()

        pltpu.sync_copy(out_v, out_hbm.at[pl.ds(half, 1), pl.ds(rb, _RPP)])

    return sc_norms


_EBS = 2048


def _epi_body(w_ref, lab_ref, tab_ref, drow_ref, s_ref,
              loss_ref, dsp_ref, acc_ref):
    i = pl.program_id(0)
    nb = pl.num_programs(0)

    @pl.when(i == 0)
    def _init():
        acc_ref[0] = 0.0
        acc_ref[1] = 0.0
        dsp_ref[...] = _softplus(drow_ref[...])

    lab = lab_ref[pl.ds(i, 1), :]
    iota = lax.broadcasted_iota(jnp.int32, (_LP, _EBS), 0)
    oh_t = (iota == lab).astype(jnp.float32)
    gt = lax.dot_general(tab_ref[...], oh_t,
                         (((0,), (0,)), ((), ())),
                         preferred_element_type=jnp.float32)
    k1 = _softplus(gt[0:1, :])
    k2 = _softplus(gt[1:2, :])
    d = _softplus(gt[2:3, :])

    s = s_ref[...]
    z1 = jnp.sqrt(s[0:1, :])
    z2 = jnp.sqrt(s[1:2, :])
    euc = z1 * k1 + z2 * k2
    acc_ref[0] += jnp.sum(jnp.maximum(euc - d, 0.0))
    acc_ref[1] += jnp.sum(jnp.maximum(d - euc, 0.0))

    @pl.when(i == nb - 1)
    def _fin():
        loss_ref[0, 0] = (w_ref[0, 0] * acc_ref[0] + acc_ref[1]) / (nb * _EBS)


def _epilogue(s_arr, labels2d, tab, drow, w_arr):
    nb = _B // _EBS
    return pl.pallas_call(
        _epi_body,
        grid=(nb,),
        in_specs=[
            pl.BlockSpec(memory_space=pltpu.SMEM),
            pl.BlockSpec((nb, _EBS), lambda i: (0, 0)),
            pl.BlockSpec((_LP, 8), lambda i: (0, 0)),
            pl.BlockSpec((1, _LP), lambda i: (0, 0)),
            pl.BlockSpec((2, _EBS), lambda i: (0, i)),
        ],
        out_specs=[
            pl.BlockSpec(memory_space=pltpu.SMEM),
            pl.BlockSpec((1, _LP), lambda i: (0, 0)),
        ],
        out_shape=[
            jax.ShapeDtypeStruct((1, 1), jnp.float32),
            jax.ShapeDtypeStruct((1, _LP), jnp.float32),
        ],
        scratch_shapes=[pltpu.SMEM((2,), jnp.float32)],
        compiler_params=pltpu.CompilerParams(
            dimension_semantics=("arbitrary",),
        ),
    )(w_arr, labels2d, tab, drow, s_arr)


def kernel(pooled_output, centroids, labels, delta, param_ab, w=1.0):
    labels = labels.astype(jnp.int32)
    cent_flat = jnp.transpose(
        centroids.reshape(_L, 2, _H), (1, 0, 2)).reshape(-1)
    sc_norms = _make_sc_norms()
    s_arr = sc_norms(pooled_output, cent_flat, labels)

    tab = jnp.zeros((_LP, 8), jnp.float32)
    tab = tab.at[:_L, 0].set(param_ab[:, 0])
    tab = tab.at[:_L, 1].set(param_ab[:, 1])
    tab = tab.at[:_L, 2].set(delta)
    drow = jnp.zeros((1, _LP), jnp.float32).at[0, :_L].set(delta)
    lab2d = labels.reshape(_B // _EBS, _EBS)
    w_arr = jnp.asarray(w, jnp.float32).reshape(1, 1)

    loss, dsp_row = _epilogue(s_arr, lab2d, tab, drow, w_arr)
    return loss[0, 0], dsp_row[0, :_L]
